# Initial kernel scaffold; baseline (speedup 1.0000x reference)
import jax
import jax.numpy as jnp
from jax.experimental import pallas as pl


def kernel(dist, sh, edge_index, sub_nodes):
    raise NotImplementedError("write your pallas kernel here")


# speedup vs baseline: 1.3575x; 1.3575x over previous
import functools

import jax
import jax.numpy as jnp
from jax.experimental import pallas as pl

N_NODES = 100000
NUM_GAUSS = 32
SH_DIM = 9
STOP = 5.0

_EDGE_BLK = 512


def _edge_body(dist_ref, sh_ref, gs_ref, gd_ref, out_ref, es_ref, ed_ref):
    gs = gs_ref[:, :]
    gd = gd_ref[:, :]
    mask = (gs >= 0) & (gd >= 0)
    mf = mask.astype(jnp.float32)

    d = dist_ref[:, :]
    step = STOP / (NUM_GAUSS - 1)
    offset = jax.lax.broadcasted_iota(
        jnp.int32, (1, NUM_GAUSS), 1).astype(jnp.float32) * step
    coeff = -0.5 / (step * step)
    t = d - offset
    ea = jnp.exp(coeff * t * t) * mf
    shm = sh_ref[:, :] * mf
    out_ref[:, :] = jnp.concatenate([ea, shm], axis=1)

    neg1 = jnp.full(gs.shape, -1, jnp.int32)
    es_ref[:, :] = jnp.where(mask, gs, neg1)
    ed_ref[:, :] = jnp.where(mask, gd, neg1)


def _edge_stage(dist, sh, gsrc, gdst):
    E = dist.shape[0]
    B = _EDGE_BLK
    grid = (E // B,)
    col = pl.BlockSpec((B, 1), lambda i: (i, 0))
    out, es, ed = pl.pallas_call(
        _edge_body,
        grid=grid,
        in_specs=[
            col,
            pl.BlockSpec((B, SH_DIM), lambda i: (i, 0)),
            col,
            col,
        ],
        out_specs=[
            pl.BlockSpec((B, NUM_GAUSS + SH_DIM), lambda i: (i, 0)),
            col,
            col,
        ],
        out_shape=[
            jax.ShapeDtypeStruct((E, NUM_GAUSS + SH_DIM), jnp.float32),
            jax.ShapeDtypeStruct((E, 1), jnp.int32),
            jax.ShapeDtypeStruct((E, 1), jnp.int32),
        ],
    )(dist.reshape(E, 1), sh, gsrc.reshape(E, 1), gdst.reshape(E, 1))
    return out, es.reshape(E), ed.reshape(E)


def kernel(dist, sh, edge_index, sub_nodes):
    E = dist.shape[0]
    n_sub = sub_nodes.shape[0]
    src = edge_index[0]
    dst = edge_index[1]

    idx_map = jnp.full((N_NODES,), -1, jnp.int32).at[sub_nodes].set(
        jnp.arange(n_sub, dtype=jnp.int32))
    a_mask = jnp.zeros((N_NODES,), dtype=bool).at[src].set(True)
    b_mask = jnp.zeros((N_NODES,), dtype=bool).at[dst].set(True)
    na = a_mask.astype(jnp.int32).sum()
    a_rank = jnp.cumsum(a_mask.astype(jnp.int32)) - 1
    b_rank = jnp.cumsum(b_mask.astype(jnp.int32)) - 1 + na

    gsrc = idx_map[src]
    gdst = idx_map[dst]
    inter_ei = jnp.stack([a_rank[src], b_rank[dst]], axis=0)

    out, es, ed = _edge_stage(dist, sh, gsrc, gdst)
    sub_ei = jnp.stack([es, ed], axis=0)
    return out, sub_ei, inter_ei


# speedup vs baseline: 5.2526x; 3.8693x over previous
import functools

import jax
import jax.numpy as jnp
from jax import lax
from jax.experimental import pallas as pl
from jax.experimental.pallas import tpu as pltpu
from jax.experimental.pallas import tpu_sc as plsc

N_NODES = 100000
NUM_GAUSS = 32
SH_DIM = 9
STOP = 5.0

_NPAD = 100352
_EDGE_BLK = 512

_NTILES = 32
_GCH = 10000


def _sc_gather_body(imap_hbm, arank_hbm, brank_hbm, src_hbm, dst_hbm,
                    gsrc_hbm, gdst_hbm, ra_hbm, rb_hbm,
                    table_v, idx_v, out_v):
    E = src_hbm.shape[0]
    epw = E // _NTILES
    wid = lax.axis_index("s") * 2 + lax.axis_index("c")
    base = wid * epw
    nch = epw // _GCH

    def one_pass(table_hbm, eidx_hbm, out_hbm):
        pltpu.sync_copy(table_hbm, table_v)

        def chunk(j, carry):
            off = base + j * _GCH
            pltpu.sync_copy(eidx_hbm.at[pl.ds(off, _GCH)], idx_v)

            def inner(i, c2):
                iv = idx_v[pl.ds(i * 16, 16)]
                out_v[pl.ds(i * 16, 16)] = plsc.load_gather(table_v, [iv])
                return c2

            lax.fori_loop(0, _GCH // 16, inner, 0, unroll=8)
            pltpu.sync_copy(out_v, out_hbm.at[pl.ds(off, _GCH)])
            return carry

        lax.fori_loop(0, nch, chunk, 0)

    one_pass(imap_hbm, src_hbm, gsrc_hbm)
    one_pass(imap_hbm, dst_hbm, gdst_hbm)
    one_pass(arank_hbm, src_hbm, ra_hbm)
    one_pass(brank_hbm, dst_hbm, rb_hbm)


def _sc_gather(imap, arank, brank, src, dst):
    E = src.shape[0]
    mesh = plsc.VectorSubcoreMesh(core_axis_name="c", subcore_axis_name="s")
    out_t = [jax.ShapeDtypeStruct((E,), jnp.int32)] * 4
    f = pl.kernel(
        _sc_gather_body,
        out_type=out_t,
        mesh=mesh,
        scratch_types=[
            pltpu.VMEM((_NPAD,), jnp.int32),
            pltpu.VMEM((_GCH,), jnp.int32),
            pltpu.VMEM((_GCH,), jnp.int32),
        ],
        compiler_params=pltpu.CompilerParams(needs_layout_passes=False),
    )
    return f(imap, arank, brank, src, dst)


def _edge_body(dist_ref, sh_ref, gs_ref, gd_ref, out_ref, es_ref, ed_ref):
    gs = gs_ref[:, :]
    gd = gd_ref[:, :]
    mask = (gs >= 0) & (gd >= 0)
    mf = mask.astype(jnp.float32)

    d = dist_ref[:, :]
    step = STOP / (NUM_GAUSS - 1)
    offset = jax.lax.broadcasted_iota(
        jnp.int32, (1, NUM_GAUSS), 1).astype(jnp.float32) * step
    coeff = -0.5 / (step * step)
    t = d - offset
    ea = jnp.exp(coeff * t * t) * mf
    shm = sh_ref[:, :] * mf
    out_ref[:, :] = jnp.concatenate([ea, shm], axis=1)

    neg1 = jnp.full(gs.shape, -1, jnp.int32)
    es_ref[:, :] = jnp.where(mask, gs, neg1)
    ed_ref[:, :] = jnp.where(mask, gd, neg1)


def _edge_stage(dist, sh, gsrc, gdst):
    E = dist.shape[0]
    B = _EDGE_BLK
    grid = (E // B,)
    col = pl.BlockSpec((B, 1), lambda i: (i, 0))
    out, es, ed = pl.pallas_call(
        _edge_body,
        grid=grid,
        in_specs=[
            col,
            pl.BlockSpec((B, SH_DIM), lambda i: (i, 0)),
            col,
            col,
        ],
        out_specs=[
            pl.BlockSpec((B, NUM_GAUSS + SH_DIM), lambda i: (i, 0)),
            col,
            col,
        ],
        out_shape=[
            jax.ShapeDtypeStruct((E, NUM_GAUSS + SH_DIM), jnp.float32),
            jax.ShapeDtypeStruct((E, 1), jnp.int32),
            jax.ShapeDtypeStruct((E, 1), jnp.int32),
        ],
    )(dist.reshape(E, 1), sh, gsrc.reshape(E, 1), gdst.reshape(E, 1))
    return out, es.reshape(E), ed.reshape(E)


def kernel(dist, sh, edge_index, sub_nodes):
    E = dist.shape[0]
    n_sub = sub_nodes.shape[0]
    src = edge_index[0]
    dst = edge_index[1]

    idx_map = jnp.full((_NPAD,), -1, jnp.int32).at[sub_nodes].set(
        jnp.arange(n_sub, dtype=jnp.int32))
    a_mask = jnp.zeros((_NPAD,), dtype=jnp.int32).at[src].set(1)
    b_mask = jnp.zeros((_NPAD,), dtype=jnp.int32).at[dst].set(1)
    na = a_mask[:N_NODES].sum()
    a_rank = jnp.cumsum(a_mask) - 1
    b_rank = jnp.cumsum(b_mask) - 1 + na

    gsrc, gdst, ra, rb = _sc_gather(idx_map, a_rank, b_rank, src, dst)
    inter_ei = jnp.stack([ra, rb], axis=0)

    out, es, ed = _edge_stage(dist, sh, gsrc, gdst)
    sub_ei = jnp.stack([es, ed], axis=0)
    return out, sub_ei, inter_ei


# speedup vs baseline: 9.6813x; 1.8431x over previous
import functools

import jax
import jax.numpy as jnp
from jax import lax
from jax.experimental import pallas as pl
from jax.experimental.pallas import tpu as pltpu
from jax.experimental.pallas import tpu_sc as plsc

N_NODES = 100000
NUM_GAUSS = 32
SH_DIM = 9
STOP = 5.0

_NPAD = 100352
_EDGE_BLK = 512

_NTILES = 32
_GCH = 10000


_ZCH = _NPAD // 16


def _sc_scatter_body(srcR, dstR, subR, svalsR, ones_hbm,
                     a0, a1, b0, b1, imap,
                     zbuf, ebuf, ebuf2, vbuf, ones_v):
    c = lax.axis_index("c")
    s = lax.axis_index("s")
    nsub_rows = subR.shape[0]

    pltpu.sync_copy(ones_hbm, ones_v)

    def zfill(k, carry):
        zbuf[pl.ds(k * 16, 16)] = jnp.zeros((16,), jnp.int32)
        return carry

    lax.fori_loop(0, _ZCH // 16, zfill, 0, unroll=8)
    sl = pl.ds(s * _ZCH, _ZCH)

    @pl.when(c == 0)
    def _():
        pltpu.sync_copy(zbuf, a0.at[sl])
        pltpu.sync_copy(zbuf, b0.at[sl])

    @pl.when(c == 1)
    def _():
        pltpu.sync_copy(zbuf, a1.at[sl])
        pltpu.sync_copy(zbuf, b1.at[sl])

    def mfill(k, carry):
        zbuf[pl.ds(k * 16, 16)] = jnp.full((16,), -1, jnp.int32)
        return carry

    lax.fori_loop(0, _ZCH // 16, mfill, 0, unroll=8)

    @pl.when(c == 0)
    def _():
        pltpu.sync_copy(zbuf, imap.at[sl])

    plsc.subcore_barrier()

    nrows_half = srcR.shape[0] // 2
    per = nrows_half // 16
    extra = nrows_half - per * 16
    row0 = c * nrows_half + per * s + jnp.minimum(s, extra)
    nrows = per + (s < extra).astype(jnp.int32)

    def srow(j, carry):
        r = row0 + j
        pltpu.sync_copy(srcR.at[pl.ds(r, 1)], ebuf)
        pltpu.sync_copy(dstR.at[pl.ds(r, 1)], ebuf2)

        @pl.when(c == 0)
        def _():
            pltpu.sync_copy(ones_v, a0.at[ebuf.at[0]])
            pltpu.sync_copy(ones_v, b0.at[ebuf2.at[0]])

        @pl.when(c == 1)
        def _():
            pltpu.sync_copy(ones_v, a1.at[ebuf.at[0]])
            pltpu.sync_copy(ones_v, b1.at[ebuf2.at[0]])

        return carry

    lax.fori_loop(0, nrows, srow, 0)

    @pl.when(c == 0)
    def _():
        def mrow(t, carry):
            r = s + t * 16

            @pl.when(r < nsub_rows)
            def _():
                pltpu.sync_copy(subR.at[pl.ds(r, 1)], ebuf)
                pltpu.sync_copy(svalsR.at[pl.ds(r, 1)], vbuf)
                pltpu.sync_copy(vbuf.at[0], imap.at[ebuf.at[0]])

            return carry

        lax.fori_loop(0, pl.cdiv(nsub_rows, 16), mrow, 0)


def _sc_scatter(src, dst, sub_nodes, n_sub_pad):
    E = src.shape[0]
    rows = E // 128
    srcR = src.reshape(rows, 128)
    dstR = dst.reshape(rows, 128)
    sub_pad = jnp.concatenate(
        [sub_nodes, jnp.full((n_sub_pad - sub_nodes.shape[0],), _NPAD - 1,
                             jnp.int32)])
    subR = sub_pad.reshape(n_sub_pad // 128, 128)
    svalsR = jnp.arange(n_sub_pad, dtype=jnp.int32).reshape(
        n_sub_pad // 128, 128)
    ones = jnp.ones((128,), jnp.int32)

    mesh = plsc.VectorSubcoreMesh(core_axis_name="c", subcore_axis_name="s")
    out_t = [jax.ShapeDtypeStruct((_NPAD,), jnp.int32)] * 5
    f = pl.kernel(
        _sc_scatter_body,
        out_type=out_t,
        mesh=mesh,
        scratch_types=[
            pltpu.VMEM((_ZCH,), jnp.int32),
            pltpu.VMEM((1, 128), jnp.int32),
            pltpu.VMEM((1, 128), jnp.int32),
            pltpu.VMEM((1, 128), jnp.int32),
            pltpu.VMEM((128,), jnp.int32),
        ],
        compiler_params=pltpu.CompilerParams(needs_layout_passes=False),
    )
    return f(srcR, dstR, subR, svalsR, ones)


def _tc_rank_body(a0, a1, b0, b1, arank_ref, brank_ref):
    R = a0.shape[0]
    af = ((a0[:, :] + a1[:, :]) > 0).astype(jnp.float32)
    bf = ((b0[:, :] + b1[:, :]) > 0).astype(jnp.float32)

    r128 = lax.broadcasted_iota(jnp.int32, (128, 128), 0)
    c128 = lax.broadcasted_iota(jnp.int32, (128, 128), 1)
    U = (r128 <= c128).astype(jnp.float32)

    rR = lax.broadcasted_iota(jnp.int32, (R, R), 0)
    cR = lax.broadcasted_iota(jnp.int32, (R, R), 1)
    L = (rR > cR).astype(jnp.float32)

    rca = jnp.dot(af, U, preferred_element_type=jnp.float32)
    rcb = jnp.dot(bf, U, preferred_element_type=jnp.float32)
    rsa = rca[:, 127:128]
    rsb = rcb[:, 127:128]
    offa = jnp.dot(L, rsa, preferred_element_type=jnp.float32)
    offb = jnp.dot(L, rsb, preferred_element_type=jnp.float32)

    na = (offa[R - 1:R, :] + rsa[R - 1:R, :]).astype(jnp.int32)

    arank_ref[:, :] = (rca + offa).astype(jnp.int32) - 1
    brank_ref[:, :] = (rcb + offb).astype(jnp.int32) - 1 + na


def _tc_ranks(a0, a1, b0, b1):
    R = _NPAD // 128
    shp = jax.ShapeDtypeStruct((R, 128), jnp.int32)
    arank, brank = pl.pallas_call(
        _tc_rank_body,
        out_shape=[shp, shp],
    )(a0.reshape(R, 128), a1.reshape(R, 128),
      b0.reshape(R, 128), b1.reshape(R, 128))
    return arank.reshape(_NPAD), brank.reshape(_NPAD)


def _sc_gather_body(imap_hbm, arank_hbm, brank_hbm, src_hbm, dst_hbm,
                    gsrc_hbm, gdst_hbm, ra_hbm, rb_hbm,
                    table_v, idx_v, out_v):
    E = src_hbm.shape[0]
    epw = E // _NTILES
    wid = lax.axis_index("s") * 2 + lax.axis_index("c")
    base = wid * epw
    nch = epw // _GCH

    def one_pass(table_hbm, eidx_hbm, out_hbm):
        pltpu.sync_copy(table_hbm, table_v)

        def chunk(j, carry):
            off = base + j * _GCH
            pltpu.sync_copy(eidx_hbm.at[pl.ds(off, _GCH)], idx_v)

            def inner(i, c2):
                iv = idx_v[pl.ds(i * 16, 16)]
                out_v[pl.ds(i * 16, 16)] = plsc.load_gather(table_v, [iv])
                return c2

            lax.fori_loop(0, _GCH // 16, inner, 0, unroll=8)
            pltpu.sync_copy(out_v, out_hbm.at[pl.ds(off, _GCH)])
            return carry

        lax.fori_loop(0, nch, chunk, 0)

    one_pass(imap_hbm, src_hbm, gsrc_hbm)
    one_pass(imap_hbm, dst_hbm, gdst_hbm)
    one_pass(arank_hbm, src_hbm, ra_hbm)
    one_pass(brank_hbm, dst_hbm, rb_hbm)


def _sc_gather(imap, arank, brank, src, dst):
    E = src.shape[0]
    mesh = plsc.VectorSubcoreMesh(core_axis_name="c", subcore_axis_name="s")
    out_t = [jax.ShapeDtypeStruct((E,), jnp.int32)] * 4
    f = pl.kernel(
        _sc_gather_body,
        out_type=out_t,
        mesh=mesh,
        scratch_types=[
            pltpu.VMEM((_NPAD,), jnp.int32),
            pltpu.VMEM((_GCH,), jnp.int32),
            pltpu.VMEM((_GCH,), jnp.int32),
        ],
        compiler_params=pltpu.CompilerParams(needs_layout_passes=False),
    )
    return f(imap, arank, brank, src, dst)


def _edge_body(dist_ref, sh_ref, gs_ref, gd_ref, out_ref, es_ref, ed_ref):
    gs = gs_ref[:, :]
    gd = gd_ref[:, :]
    mask = (gs >= 0) & (gd >= 0)
    mf = mask.astype(jnp.float32)

    d = dist_ref[:, :]
    step = STOP / (NUM_GAUSS - 1)
    offset = jax.lax.broadcasted_iota(
        jnp.int32, (1, NUM_GAUSS), 1).astype(jnp.float32) * step
    coeff = -0.5 / (step * step)
    t = d - offset
    ea = jnp.exp(coeff * t * t) * mf
    shm = sh_ref[:, :] * mf
    out_ref[:, :] = jnp.concatenate([ea, shm], axis=1)

    neg1 = jnp.full(gs.shape, -1, jnp.int32)
    es_ref[:, :] = jnp.where(mask, gs, neg1)
    ed_ref[:, :] = jnp.where(mask, gd, neg1)


def _edge_stage(dist, sh, gsrc, gdst):
    E = dist.shape[0]
    B = _EDGE_BLK
    grid = (E // B,)
    col = pl.BlockSpec((B, 1), lambda i: (i, 0))
    out, es, ed = pl.pallas_call(
        _edge_body,
        grid=grid,
        in_specs=[
            col,
            pl.BlockSpec((B, SH_DIM), lambda i: (i, 0)),
            col,
            col,
        ],
        out_specs=[
            pl.BlockSpec((B, NUM_GAUSS + SH_DIM), lambda i: (i, 0)),
            col,
            col,
        ],
        out_shape=[
            jax.ShapeDtypeStruct((E, NUM_GAUSS + SH_DIM), jnp.float32),
            jax.ShapeDtypeStruct((E, 1), jnp.int32),
            jax.ShapeDtypeStruct((E, 1), jnp.int32),
        ],
    )(dist.reshape(E, 1), sh, gsrc.reshape(E, 1), gdst.reshape(E, 1))
    return out, es.reshape(E), ed.reshape(E)


def kernel(dist, sh, edge_index, sub_nodes):
    E = dist.shape[0]
    n_sub = sub_nodes.shape[0]
    src = edge_index[0]
    dst = edge_index[1]

    n_sub_pad = ((n_sub + 127) // 128) * 128
    a0, a1, b0, b1, idx_map = _sc_scatter(src, dst, sub_nodes, n_sub_pad)
    a_rank, b_rank = _tc_ranks(a0, a1, b0, b1)

    gsrc, gdst, ra, rb = _sc_gather(idx_map, a_rank, b_rank, src, dst)
    inter_ei = jnp.stack([ra, rb], axis=0)

    out, es, ed = _edge_stage(dist, sh, gsrc, gdst)
    sub_ei = jnp.stack([es, ed], axis=0)
    return out, sub_ei, inter_ei
